# baseline (speedup 1.0000x reference)
import jax
import jax.numpy as jnp
from jax.experimental import pallas as pl
from jax.experimental.pallas import tpu as pltpu

DEPTH_ = 100000
BATCH_ = 1024

BLOCK_R = 8
N_BUF = 8
GRID_ = BATCH_ // BLOCK_R


def _onehot_block(idx_ref, col_ref, out_hbm, scratch, sems):
    i = pl.program_id(0)
    slot = jax.lax.rem(i, N_BUF)

    @pl.when(i >= N_BUF)
    def _():
        rows_prev = (i - N_BUF) * BLOCK_R
        pltpu.make_async_copy(
            scratch.at[slot],
            out_hbm.at[pl.ds(rows_prev, BLOCK_R), :],
            sems.at[slot],
        ).wait()

    idx = idx_ref[...]
    col = col_ref[...]
    scratch.at[slot][...] = (col == idx).astype(jnp.float32)
    pltpu.make_async_copy(
        scratch.at[slot],
        out_hbm.at[pl.ds(i * BLOCK_R, BLOCK_R), :],
        sems.at[slot],
    ).start()

    @pl.when(i == GRID_ - 1)
    def _():
        for s in range(N_BUF):
            step = GRID_ - N_BUF + s
            pltpu.make_async_copy(
                scratch.at[s],
                out_hbm.at[pl.ds(step * BLOCK_R, BLOCK_R), :],
                sems.at[s],
            ).wait()


def kernel(input):
    idx = input.astype(jnp.int32)
    col = jax.lax.broadcasted_iota(jnp.int32, (1, DEPTH_), 1)
    out = pl.pallas_call(
        _onehot_block,
        grid=(GRID_,),
        in_specs=[
            pl.BlockSpec((BLOCK_R, 1), lambda i: (i, 0)),
            pl.BlockSpec((1, DEPTH_), lambda i: (0, 0)),
        ],
        out_specs=pl.BlockSpec(memory_space=pl.ANY),
        out_shape=jax.ShapeDtypeStruct((BATCH_, DEPTH_), jnp.float32),
        scratch_shapes=[
            pltpu.VMEM((N_BUF, BLOCK_R, DEPTH_), jnp.float32),
            pltpu.SemaphoreType.DMA((N_BUF,)),
        ],
    )(idx, col)
    return out
